# baseline (speedup 1.0000x reference)
import functools

import jax
import jax.numpy as jnp
from jax import lax
from jax.experimental import pallas as pl
from jax.experimental.pallas import tpu as pltpu
from jax.experimental.pallas import tpu_sc as plsc

_VOCAB = 100000
_BATCH = 64
_TOP_P = 0.9
_L = 16
_U = 5
_XI_CHUNK = 20000
_HIST = 512 * _L

_LN2 = 0.6931471805599453
_SQRT2 = 1.4142135623730951
_LEVELS = ((21, 512, None), (12, 512, 1 << 21), (3, 512, 1 << 12), (0, 8, 1 << 3))


def _neg_log(x):
    u = plsc.bitcast(x, jnp.uint32)
    ex = (u >> jnp.uint32(23)).astype(jnp.int32) - 127
    m = plsc.bitcast(
        (u & jnp.uint32(0x007FFFFF)) | jnp.uint32(0x3F800000), jnp.float32
    )
    big = m > jnp.float32(_SQRT2)
    r = jnp.where(big, m * jnp.float32(0.5), m)
    n = (ex + big.astype(jnp.int32)).astype(jnp.float32)
    s = (r - jnp.float32(1.0)) / (r + jnp.float32(1.0))
    s2 = s * s
    p = jnp.float32(1.0 / 9.0)
    for c in (1.0 / 7.0, 1.0 / 5.0, 1.0 / 3.0, 1.0):
        p = p * s2 + jnp.float32(c)
    lnr = jnp.float32(2.0) * s * p
    return -(n * jnp.float32(_LN2) + lnr)


def _select_row(r, logits_hbm, xi_hbm, e_v, hist, xi_buf):
    lane = lax.iota(jnp.int32, _L)
    lane_blk = lane << 9
    n_grp = _VOCAB // (_L * _U)

    pltpu.sync_copy(logits_hbm.at[pl.ds(r * _VOCAB, _VOCAB)], e_v)

    def max_body(i, ms):
        return tuple(
            jnp.maximum(ms[s], e_v[pl.ds((i * _U + s) * _L, _L)])
            for s in range(_U)
        )

    ms = lax.fori_loop(
        0, n_grp, max_body,
        tuple(jnp.full((_L,), -jnp.inf, jnp.float32) for _ in range(_U)),
    )
    mx = ms[0]
    for s in range(1, _U):
        mx = jnp.maximum(mx, ms[s])
    m_vec = jnp.full((_L,), jnp.max(mx), jnp.float32)

    def exp_body(i, zs):
        out = []
        for s in range(_U):
            sl = pl.ds((i * _U + s) * _L, _L)
            ev = jnp.exp(e_v[sl] - m_vec)
            e_v[sl] = ev
            u = plsc.bitcast(ev, jnp.uint32)
            idx = lane_blk | (u >> jnp.uint32(21)).astype(jnp.int32)
            plsc.addupdate_scatter(hist, [idx], ev)
            out.append(zs[s] + ev)
        return tuple(out)

    zs = lax.fori_loop(
        0, n_grp, exp_body,
        tuple(jnp.zeros((_L,), jnp.float32) for _ in range(_U)),
    )
    zv = zs[0]
    for s in range(1, _U):
        zv = zv + zs[s]
    budget = jnp.float32(_TOP_P) * jnp.sum(zv)

    base = jnp.uint32(0)
    for sh, width, span in _LEVELS:
        if span is not None:
            base_vec = jnp.full((_L,), base, jnp.uint32)
            span_u = jnp.uint32(span)

            def hist_body(i, _):
                for s in range(_U):
                    ev = e_v[pl.ds((i * _U + s) * _L, _L)]
                    rel = plsc.bitcast(ev, jnp.uint32) - base_vec
                    ok = rel < span_u
                    binv = (rel >> jnp.uint32(sh)).astype(jnp.int32)
                    idx = jnp.where(ok, lane_blk | binv, lane_blk)
                    plsc.addupdate_scatter(hist, [idx], ev, mask=ok)
                return 0

            lax.fori_loop(0, n_grp, hist_body, 0)

        n_ch = (width + _L - 1) // _L
        budget_vec = jnp.full((_L,), budget, jnp.float32)
        zero16 = jnp.zeros((_L,), jnp.float32)

        def scan_body(j, carry):
            acc, found, base_bin = carry
            off = (n_ch - 1 - j) * _L
            mg = zero16
            for lane_l in range(_L):
                sl = pl.ds(lane_l * 512 + off, _L)
                mg = mg + hist[sl]
                hist[sl] = zero16
            t_sum = jnp.sum(mg)
            cum = plsc.cumsum(mg)
            c_b = jnp.full((_L,), acc, jnp.float32) + (t_sum - cum + mg)
            nt = plsc.all_reduce_population_count(c_b >= budget_vec)[0]
            cross = jnp.logical_and(jnp.logical_not(found), nt > 0)
            bs = nt - 1
            above = jnp.sum(
                jnp.where(lane > jnp.full((_L,), bs, jnp.int32), mg, zero16)
            )
            acc = jnp.where(
                found, acc, jnp.where(cross, acc + above, acc + t_sum)
            )
            base_bin = jnp.where(cross, off + bs, base_bin)
            return acc, jnp.logical_or(found, cross), base_bin

        acc, found, base_bin = lax.fori_loop(
            0, n_ch, scan_body, (jnp.float32(0.0), False, jnp.int32(0))
        )
        budget = budget - acc
        base = base + (base_bin.astype(jnp.uint32) << jnp.uint32(sh))

    t_vec = jnp.full((_L,), base, jnp.uint32)
    et_vec = plsc.bitcast(t_vec, jnp.float32)
    r_vec = jnp.full((_L,), budget, jnp.float32)

    n_sgrp = _XI_CHUNK // (_L * _U)

    def chunk_body(c, carry):
        pltpu.sync_copy(
            xi_hbm.at[pl.ds(r * _VOCAB + c * _XI_CHUNK, _XI_CHUNK)], xi_buf
        )

        def score_body(i, carry):
            bE, be, bi, gidx, cnt = carry
            evs, us, Es, eqs, pcs = [], [], [], [], []
            for s in range(_U):
                ev = e_v[pl.ds(c * _XI_CHUNK + (i * _U + s) * _L, _L)]
                u = plsc.bitcast(ev, jnp.uint32)
                E = _neg_log(xi_buf[pl.ds((i * _U + s) * _L, _L)])
                eq = u == t_vec
                evs.append(ev)
                us.append(u)
                Es.append(E)
                eqs.append(eq)
                pcs.append(plsc.all_reduce_population_count(eq))
            pc_tot = pcs[0]
            for s in range(1, _U):
                pc_tot = pc_tot + pcs[s]

            def tie_path():
                incs, c0 = [], cnt
                for s in range(_U):
                    pfx = plsc.cumsum(jnp.where(eqs[s], 1, 0).astype(jnp.int32))
                    rank_f = (c0 + pfx - 1).astype(jnp.float32)
                    incs.append(
                        jnp.logical_or(
                            us[s] > t_vec,
                            jnp.logical_and(eqs[s], rank_f * et_vec < r_vec),
                        )
                    )
                    c0 = c0 + pcs[s]
                return tuple(incs) + (c0,)

            def fast_path():
                return tuple(us[s] > t_vec for s in range(_U)) + (cnt,)

            res = lax.cond(pc_tot[0] > 0, tie_path, fast_path)
            incs, cnt = res[:_U], res[_U]

            nbE, nbe, nbi = list(bE), list(be), list(bi)
            for s in range(_U):
                better = jnp.logical_and(
                    incs[s], Es[s] * be[s] < bE[s] * evs[s]
                )
                nbE[s] = jnp.where(better, Es[s], bE[s])
                nbe[s] = jnp.where(better, evs[s], be[s])
                nbi[s] = jnp.where(better, gidx[s], bi[s])
            ngidx = tuple(g + _L * _U for g in gidx)
            return tuple(nbE), tuple(nbe), tuple(nbi), ngidx, cnt

        bE, be, bi, gidx, cnt = lax.fori_loop(0, n_sgrp, score_body, carry[:5])
        gidx = tuple(
            jnp.full((_L,), (c + 1) * _XI_CHUNK + s * _L, jnp.int32) + lane
            for s in range(_U)
        )
        return bE, be, bi, gidx, cnt

    init = (
        tuple(jnp.full((_L,), jnp.inf, jnp.float32) for _ in range(_U)),
        tuple(jnp.full((_L,), 1.0, jnp.float32) for _ in range(_U)),
        tuple(jnp.zeros((_L,), jnp.int32) for _ in range(_U)),
        tuple(jnp.full((_L,), s * _L, jnp.int32) + lane for s in range(_U)),
        jnp.zeros((_L,), jnp.int32),
    )
    bE, be, bi, _, _ = lax.fori_loop(0, _VOCAB // _XI_CHUNK, chunk_body, init)

    aE, ae, ai = bE[0], be[0], bi[0]
    for s in range(1, _U):
        better = bE[s] * ae < aE * be[s]
        aE = jnp.where(better, bE[s], aE)
        ae = jnp.where(better, be[s], ae)
        ai = jnp.where(better, bi[s], ai)
    sc = aE / ae
    smin = jnp.min(sc)
    return jnp.min(jnp.where(sc == smin, ai, jnp.int32(2**31 - 1)))


def _sc_sample(logits, xi):
    mesh = plsc.VectorSubcoreMesh(core_axis_name="c", subcore_axis_name="s")

    @functools.partial(
        pl.kernel,
        out_type=jax.ShapeDtypeStruct((_BATCH * _VOCAB,), jnp.float32),
        mesh=mesh,
        scratch_types=[
            pltpu.VMEM((_VOCAB,), jnp.float32),
            pltpu.VMEM((_HIST,), jnp.float32),
            pltpu.VMEM((_XI_CHUNK,), jnp.float32),
            pltpu.VMEM((_L,), jnp.float32),
            pltpu.SemaphoreType.DMA,
            pltpu.SemaphoreType.DMA,
        ],
        compiler_params=pltpu.CompilerParams(needs_layout_passes=False),
    )
    def run(logits_hbm, xi_hbm, out_hbm, e_v, hist, xi_buf, patch_v, sem0, sem1):
        lane = lax.iota(jnp.int32, _L)
        sid = lax.axis_index("s")
        cid = lax.axis_index("c")
        wid = cid * 16 + sid
        r0 = wid * 2
        sc_words = 32 * _VOCAB
        main_w = 1562 * 128
        off0 = pl.multiple_of(cid * sc_words + sid * main_w, 128)
        sl = pl.ds(off0, main_w)
        cps = [pltpu.async_copy(logits_hbm.at[sl], out_hbm.at[sl], sem0)]
        tail_off = pl.multiple_of(cid * sc_words + 16 * main_w, 128)
        tail_sl = pl.ds(tail_off, sc_words - 16 * main_w)

        @pl.when(sid == 0)
        def _():
            pltpu.async_copy(logits_hbm.at[tail_sl], out_hbm.at[tail_sl], sem1).wait()

        def zero_body(i, _):
            hist[pl.ds(i * _L, _L)] = jnp.zeros((_L,), jnp.float32)
            return 0

        lax.fori_loop(0, _HIST // _L, zero_body, 0)

        wins = [
            _select_row(r0 + rr, logits_hbm, xi_hbm, e_v, hist, xi_buf)
            for rr in range(2)
        ]
        for cp in cps:
            cp.wait()
        plsc.subcore_barrier()
        for rr in range(2):
            win = wins[rr]
            off = pl.multiple_of((r0 + rr) * _VOCAB + ((win >> 4) << 4), 16)
            seg = out_hbm.at[pl.ds(off, _L)]
            pltpu.sync_copy(seg, patch_v)
            low_vec = jnp.full((_L,), win & 15, jnp.int32)
            patch_v[...] = patch_v[...] + jnp.where(
                lane == low_vec, jnp.float32(50.0), jnp.float32(0.0)
            )
            pltpu.sync_copy(patch_v, seg)

    return run(logits.reshape(-1), xi.reshape(-1))


def kernel(input_ids, logits, xi):
    del input_ids
    out = _sc_sample(logits, xi)
    return out.reshape(_BATCH, _VOCAB)


# speedup vs baseline: 1.8349x; 1.8349x over previous
import functools

import jax
import jax.numpy as jnp
from jax import lax
from jax.experimental import pallas as pl
from jax.experimental.pallas import tpu as pltpu
from jax.experimental.pallas import tpu_sc as plsc

_VOCAB = 100000
_BATCH = 64
_TOP_P = 0.9
_L = 16
_U = 5
_XI_CHUNK = 20000
_HIST = 512 * _L

_LN2 = 0.6931471805599453
_SQRT2 = 1.4142135623730951
_LEVELS = ((21, 512, None), (12, 512, 1 << 21), (3, 512, 1 << 12), (0, 8, 1 << 3))


def _neg_log(x):
    u = plsc.bitcast(x, jnp.uint32)
    ex = (u >> jnp.uint32(23)).astype(jnp.int32) - 127
    m = plsc.bitcast(
        (u & jnp.uint32(0x007FFFFF)) | jnp.uint32(0x3F800000), jnp.float32
    )
    big = m > jnp.float32(_SQRT2)
    r = jnp.where(big, m * jnp.float32(0.5), m)
    n = (ex + big.astype(jnp.int32)).astype(jnp.float32)
    s = (r - jnp.float32(1.0)) / (r + jnp.float32(1.0))
    s2 = s * s
    p = jnp.float32(1.0 / 9.0)
    for c in (1.0 / 7.0, 1.0 / 5.0, 1.0 / 3.0, 1.0):
        p = p * s2 + jnp.float32(c)
    lnr = jnp.float32(2.0) * s * p
    return -(n * jnp.float32(_LN2) + lnr)


def _select_row(r, logits_hbm, xi_hbm, e_v, hist, xi_buf):
    lane = lax.iota(jnp.int32, _L)
    lane_blk = lane << 9
    n_grp = _VOCAB // (_L * _U)

    pltpu.sync_copy(logits_hbm.at[pl.ds(r * _VOCAB, _VOCAB)], e_v)

    def max_body(i, ms):
        return tuple(
            jnp.maximum(ms[s], e_v[pl.ds((i * _U + s) * _L, _L)])
            for s in range(_U)
        )

    ms = lax.fori_loop(
        0, n_grp, max_body,
        tuple(jnp.full((_L,), -jnp.inf, jnp.float32) for _ in range(_U)),
    )
    mx = ms[0]
    for s in range(1, _U):
        mx = jnp.maximum(mx, ms[s])
    m_vec = jnp.full((_L,), jnp.max(mx), jnp.float32)

    def exp_body(i, zs):
        out = []
        for s in range(_U):
            sl = pl.ds((i * _U + s) * _L, _L)
            ev = jnp.exp(e_v[sl] - m_vec)
            e_v[sl] = ev
            u = plsc.bitcast(ev, jnp.uint32)
            idx = lane_blk | (u >> jnp.uint32(21)).astype(jnp.int32)
            plsc.addupdate_scatter(hist, [idx], ev)
            out.append(zs[s] + ev)
        return tuple(out)

    zs = lax.fori_loop(
        0, n_grp, exp_body,
        tuple(jnp.zeros((_L,), jnp.float32) for _ in range(_U)),
    )
    zv = zs[0]
    for s in range(1, _U):
        zv = zv + zs[s]
    budget = jnp.float32(_TOP_P) * jnp.sum(zv)

    base = jnp.uint32(0)
    for sh, width, span in _LEVELS:
        if span is not None:
            base_vec = jnp.full((_L,), base, jnp.uint32)
            span_u = jnp.uint32(span)

            def hist_body(i, _):
                for s in range(_U):
                    ev = e_v[pl.ds((i * _U + s) * _L, _L)]
                    rel = plsc.bitcast(ev, jnp.uint32) - base_vec
                    ok = rel < span_u
                    binv = (rel >> jnp.uint32(sh)).astype(jnp.int32)
                    idx = jnp.where(ok, lane_blk | binv, lane_blk)
                    plsc.addupdate_scatter(hist, [idx], ev, mask=ok)
                return 0

            lax.fori_loop(0, n_grp, hist_body, 0)

        n_ch = (width + _L - 1) // _L
        budget_vec = jnp.full((_L,), budget, jnp.float32)
        zero16 = jnp.zeros((_L,), jnp.float32)

        def scan_body(j, carry):
            acc, found, base_bin = carry
            off = (n_ch - 1 - j) * _L
            mg = zero16
            for lane_l in range(_L):
                sl = pl.ds(lane_l * 512 + off, _L)
                mg = mg + hist[sl]
                hist[sl] = zero16
            t_sum = jnp.sum(mg)
            cum = plsc.cumsum(mg)
            c_b = jnp.full((_L,), acc, jnp.float32) + (t_sum - cum + mg)
            nt = plsc.all_reduce_population_count(c_b >= budget_vec)[0]
            cross = jnp.logical_and(jnp.logical_not(found), nt > 0)
            bs = nt - 1
            above = jnp.sum(
                jnp.where(lane > jnp.full((_L,), bs, jnp.int32), mg, zero16)
            )
            acc = jnp.where(
                found, acc, jnp.where(cross, acc + above, acc + t_sum)
            )
            base_bin = jnp.where(cross, off + bs, base_bin)
            return acc, jnp.logical_or(found, cross), base_bin

        acc, found, base_bin = lax.fori_loop(
            0, n_ch, scan_body, (jnp.float32(0.0), False, jnp.int32(0))
        )
        budget = budget - acc
        base = base + (base_bin.astype(jnp.uint32) << jnp.uint32(sh))

    t_vec = jnp.full((_L,), base, jnp.uint32)
    et_vec = plsc.bitcast(t_vec, jnp.float32)
    r_vec = jnp.full((_L,), budget, jnp.float32)

    n_sgrp = _XI_CHUNK // (_L * _U)

    def chunk_body(c, carry):
        pltpu.sync_copy(
            xi_hbm.at[pl.ds(r * _VOCAB + c * _XI_CHUNK, _XI_CHUNK)], xi_buf
        )

        def score_body(i, carry):
            bE, be, bi, gidx, cnt = carry
            evs, us, Es, eqs, pcs = [], [], [], [], []
            for s in range(_U):
                ev = e_v[pl.ds(c * _XI_CHUNK + (i * _U + s) * _L, _L)]
                u = plsc.bitcast(ev, jnp.uint32)
                E = _neg_log(xi_buf[pl.ds((i * _U + s) * _L, _L)])
                eq = u == t_vec
                evs.append(ev)
                us.append(u)
                Es.append(E)
                eqs.append(eq)
                pcs.append(plsc.all_reduce_population_count(eq))
            pc_tot = pcs[0]
            for s in range(1, _U):
                pc_tot = pc_tot + pcs[s]

            def tie_path():
                incs, c0 = [], cnt
                for s in range(_U):
                    pfx = plsc.cumsum(jnp.where(eqs[s], 1, 0).astype(jnp.int32))
                    rank_f = (c0 + pfx - 1).astype(jnp.float32)
                    incs.append(
                        jnp.logical_or(
                            us[s] > t_vec,
                            jnp.logical_and(eqs[s], rank_f * et_vec < r_vec),
                        )
                    )
                    c0 = c0 + pcs[s]
                return tuple(incs) + (c0,)

            def fast_path():
                return tuple(us[s] > t_vec for s in range(_U)) + (cnt,)

            res = lax.cond(pc_tot[0] > 0, tie_path, fast_path)
            incs, cnt = res[:_U], res[_U]

            nbE, nbe, nbi = list(bE), list(be), list(bi)
            for s in range(_U):
                better = jnp.logical_and(
                    incs[s], Es[s] * be[s] < bE[s] * evs[s]
                )
                nbE[s] = jnp.where(better, Es[s], bE[s])
                nbe[s] = jnp.where(better, evs[s], be[s])
                nbi[s] = jnp.where(better, gidx[s], bi[s])
            ngidx = tuple(g + _L * _U for g in gidx)
            return tuple(nbE), tuple(nbe), tuple(nbi), ngidx, cnt

        bE, be, bi, gidx, cnt = lax.fori_loop(0, n_sgrp, score_body, carry[:5])
        gidx = tuple(
            jnp.full((_L,), (c + 1) * _XI_CHUNK + s * _L, jnp.int32) + lane
            for s in range(_U)
        )
        return bE, be, bi, gidx, cnt

    init = (
        tuple(jnp.full((_L,), jnp.inf, jnp.float32) for _ in range(_U)),
        tuple(jnp.full((_L,), 1.0, jnp.float32) for _ in range(_U)),
        tuple(jnp.zeros((_L,), jnp.int32) for _ in range(_U)),
        tuple(jnp.full((_L,), s * _L, jnp.int32) + lane for s in range(_U)),
        jnp.zeros((_L,), jnp.int32),
    )
    bE, be, bi, _, _ = lax.fori_loop(0, _VOCAB // _XI_CHUNK, chunk_body, init)

    aE, ae, ai = bE[0], be[0], bi[0]
    for s in range(1, _U):
        better = bE[s] * ae < aE * be[s]
        aE = jnp.where(better, bE[s], aE)
        ae = jnp.where(better, be[s], ae)
        ai = jnp.where(better, bi[s], ai)
    sc = aE / ae
    smin = jnp.min(sc)
    return jnp.min(jnp.where(sc == smin, ai, jnp.int32(2**31 - 1)))


def _sc_sample(logits, xi):
    mesh = plsc.VectorSubcoreMesh(core_axis_name="c", subcore_axis_name="s")

    @functools.partial(
        pl.kernel,
        out_type=jax.ShapeDtypeStruct((_BATCH * _L,), jnp.int32),
        mesh=mesh,
        scratch_types=[
            pltpu.VMEM((_VOCAB,), jnp.float32),
            pltpu.VMEM((_HIST,), jnp.float32),
            pltpu.VMEM((_XI_CHUNK,), jnp.float32),
            pltpu.VMEM((_L,), jnp.int32),
        ],
        compiler_params=pltpu.CompilerParams(needs_layout_passes=False),
    )
    def run(logits_hbm, xi_hbm, out_hbm, e_v, hist, xi_buf, tok_v):
        sid = lax.axis_index("s")
        cid = lax.axis_index("c")
        wid = cid * 16 + sid
        r0 = wid * 2

        def zero_body(i, _):
            hist[pl.ds(i * _L, _L)] = jnp.zeros((_L,), jnp.float32)
            return 0

        lax.fori_loop(0, _HIST // _L, zero_body, 0)

        for rr in range(2):
            win = _select_row(r0 + rr, logits_hbm, xi_hbm, e_v, hist, xi_buf)
            tok_v[...] = jnp.full((_L,), win, jnp.int32)
            pltpu.sync_copy(tok_v, out_hbm.at[pl.ds((r0 + rr) * _L, _L)])

    return run(logits.reshape(-1), xi.reshape(-1))


def _tc_finish_body(tok_ref, logits_ref, out_ref):
    col = lax.broadcasted_iota(jnp.int32, (1, _VOCAB), 1)
    blk = pl.program_id(0)
    for j in range(8):
        tok = tok_ref[blk * 8 + j]
        row = logits_ref[pl.ds(j, 1), :]
        out_ref[pl.ds(j, 1), :] = jnp.where(col == tok, row + jnp.float32(50.0), row)


def _tc_finish(logits, tokens):
    return pl.pallas_call(
        _tc_finish_body,
        grid=(_BATCH // 8,),
        in_specs=[
            pl.BlockSpec(memory_space=pltpu.SMEM),
            pl.BlockSpec((8, _VOCAB), lambda i: (i, 0)),
        ],
        out_specs=pl.BlockSpec((8, _VOCAB), lambda i: (i, 0)),
        out_shape=jax.ShapeDtypeStruct((_BATCH, _VOCAB), jnp.float32),
    )(tokens, logits)


def kernel(input_ids, logits, xi):
    del input_ids
    toks = _sc_sample(logits, xi)
    return _tc_finish(logits, toks.reshape(_BATCH, _L)[:, 0])


# speedup vs baseline: 3.8897x; 2.1199x over previous
import functools

import jax
import jax.numpy as jnp
from jax import lax
from jax.experimental import pallas as pl
from jax.experimental.pallas import tpu as pltpu
from jax.experimental.pallas import tpu_sc as plsc

_VOCAB = 100000
_BATCH = 64
_TOP_P = 0.9
_L = 16
_U = 5
_XI_CHUNK = 20000
_HIST = 512 * _L

_LN2 = 0.6931471805599453
_SQRT2 = 1.4142135623730951
_LEVELS = ((21, 512, None), (12, 512, 1 << 21), (3, 512, 1 << 12), (0, 8, 1 << 3))


def _neg_log(x):
    u = plsc.bitcast(x, jnp.uint32)
    ex = (u >> jnp.uint32(23)).astype(jnp.int32) - 127
    m = plsc.bitcast(
        (u & jnp.uint32(0x007FFFFF)) | jnp.uint32(0x3F800000), jnp.float32
    )
    big = m > jnp.float32(_SQRT2)
    r = jnp.where(big, m * jnp.float32(0.5), m)
    n = (ex + big.astype(jnp.int32)).astype(jnp.float32)
    s = (r - jnp.float32(1.0)) / (r + jnp.float32(1.0))
    s2 = s * s
    p = jnp.float32(1.0 / 9.0)
    for c in (1.0 / 7.0, 1.0 / 5.0, 1.0 / 3.0, 1.0):
        p = p * s2 + jnp.float32(c)
    lnr = jnp.float32(2.0) * s * p
    return -(n * jnp.float32(_LN2) + lnr)


def _select_row(r, logits_hbm, xi_hbm, e_v, hist, xi_buf):
    lane = lax.iota(jnp.int32, _L)
    lane_blk = lane << 9
    n_grp = _VOCAB // (_L * _U)

    pltpu.sync_copy(logits_hbm.at[pl.ds(r * _VOCAB, _VOCAB)], e_v)

    @plsc.parallel_loop(
        0, n_grp, 1,
        carry=tuple(jnp.full((_L,), -jnp.inf, jnp.float32) for _ in range(_U)),
    )
    def ms(i, ms_c):
        return tuple(
            jnp.maximum(ms_c[s], e_v[pl.ds((i * _U + s) * _L, _L)])
            for s in range(_U)
        )
    mx = ms[0]
    for s in range(1, _U):
        mx = jnp.maximum(mx, ms[s])
    m_vec = jnp.full((_L,), jnp.max(mx), jnp.float32)

    @plsc.parallel_loop(
        0, n_grp, 1,
        carry=tuple(jnp.zeros((_L,), jnp.float32) for _ in range(_U)),
    )
    def zs(i, zs_c):
        out = []
        for s in range(_U):
            sl = pl.ds((i * _U + s) * _L, _L)
            ev = jnp.exp(e_v[sl] - m_vec)
            e_v[sl] = ev
            u = plsc.bitcast(ev, jnp.uint32)
            idx = lane_blk | (u >> jnp.uint32(21)).astype(jnp.int32)
            plsc.addupdate_scatter(hist, [idx], ev)
            out.append(zs_c[s] + ev)
        return tuple(out)
    zv = zs[0]
    for s in range(1, _U):
        zv = zv + zs[s]
    budget = jnp.float32(_TOP_P) * jnp.sum(zv)

    base = jnp.uint32(0)
    for sh, width, span in _LEVELS:
        if span is not None:
            base_vec = jnp.full((_L,), base, jnp.uint32)
            span_u = jnp.uint32(span)

            @plsc.parallel_loop(0, n_grp, 1)
            def _(i):
                for s in range(_U):
                    ev = e_v[pl.ds((i * _U + s) * _L, _L)]
                    rel = plsc.bitcast(ev, jnp.uint32) - base_vec
                    ok = rel < span_u
                    binv = (rel >> jnp.uint32(sh)).astype(jnp.int32)
                    idx = jnp.where(ok, lane_blk | binv, lane_blk)
                    plsc.addupdate_scatter(hist, [idx], ev, mask=ok)

        n_ch = (width + _L - 1) // _L
        budget_vec = jnp.full((_L,), budget, jnp.float32)
        zero16 = jnp.zeros((_L,), jnp.float32)

        def scan_body(j, carry):
            acc, found, base_bin = carry
            off = (n_ch - 1 - j) * _L
            mg = zero16
            for lane_l in range(_L):
                sl = pl.ds(lane_l * 512 + off, _L)
                mg = mg + hist[sl]
                hist[sl] = zero16
            t_sum = jnp.sum(mg)
            cum = plsc.cumsum(mg)
            c_b = jnp.full((_L,), acc, jnp.float32) + (t_sum - cum + mg)
            nt = plsc.all_reduce_population_count(c_b >= budget_vec)[0]
            cross = jnp.logical_and(jnp.logical_not(found), nt > 0)
            bs = nt - 1
            above = jnp.sum(
                jnp.where(lane > jnp.full((_L,), bs, jnp.int32), mg, zero16)
            )
            acc = jnp.where(
                found, acc, jnp.where(cross, acc + above, acc + t_sum)
            )
            base_bin = jnp.where(cross, off + bs, base_bin)
            return acc, jnp.logical_or(found, cross), base_bin

        acc, found, base_bin = lax.fori_loop(
            0, n_ch, scan_body, (jnp.float32(0.0), False, jnp.int32(0))
        )
        budget = budget - acc
        base = base + (base_bin.astype(jnp.uint32) << jnp.uint32(sh))

    t_vec = jnp.full((_L,), base, jnp.uint32)
    et_vec = plsc.bitcast(t_vec, jnp.float32)
    r_vec = jnp.full((_L,), budget, jnp.float32)

    n_sgrp = _XI_CHUNK // (_L * _U)

    def chunk_body(c, carry):
        pltpu.sync_copy(
            xi_hbm.at[pl.ds(r * _VOCAB + c * _XI_CHUNK, _XI_CHUNK)], xi_buf
        )

        @plsc.parallel_loop(0, n_sgrp, 1, carry=carry[:5])
        def score_res(i, carry):
            bE, be, bi, gidx, cnt = carry
            evs, us, Es, eqs, pcs = [], [], [], [], []
            for s in range(_U):
                ev = e_v[pl.ds(c * _XI_CHUNK + (i * _U + s) * _L, _L)]
                u = plsc.bitcast(ev, jnp.uint32)
                E = _neg_log(xi_buf[pl.ds((i * _U + s) * _L, _L)])
                eq = u == t_vec
                evs.append(ev)
                us.append(u)
                Es.append(E)
                eqs.append(eq)
                pcs.append(plsc.all_reduce_population_count(eq))
            pc_tot = pcs[0]
            for s in range(1, _U):
                pc_tot = pc_tot + pcs[s]

            def tie_path():
                incs, c0 = [], cnt
                for s in range(_U):
                    pfx = plsc.cumsum(jnp.where(eqs[s], 1, 0).astype(jnp.int32))
                    rank_f = (c0 + pfx - 1).astype(jnp.float32)
                    incs.append(
                        jnp.logical_or(
                            us[s] > t_vec,
                            jnp.logical_and(eqs[s], rank_f * et_vec < r_vec),
                        )
                    )
                    c0 = c0 + pcs[s]
                return tuple(incs) + (c0,)

            def fast_path():
                return tuple(us[s] > t_vec for s in range(_U)) + (cnt,)

            res = lax.cond(pc_tot[0] > 0, tie_path, fast_path)
            incs, cnt = res[:_U], res[_U]

            nbE, nbe, nbi = list(bE), list(be), list(bi)
            for s in range(_U):
                better = jnp.logical_and(
                    incs[s], Es[s] * be[s] < bE[s] * evs[s]
                )
                nbE[s] = jnp.where(better, Es[s], bE[s])
                nbe[s] = jnp.where(better, evs[s], be[s])
                nbi[s] = jnp.where(better, gidx[s], bi[s])
            ngidx = tuple(g + _L * _U for g in gidx)
            return tuple(nbE), tuple(nbe), tuple(nbi), ngidx, cnt

        bE, be, bi, gidx, cnt = score_res
        gidx = tuple(
            jnp.full((_L,), (c + 1) * _XI_CHUNK + s * _L, jnp.int32) + lane
            for s in range(_U)
        )
        return bE, be, bi, gidx, cnt

    init = (
        tuple(jnp.full((_L,), jnp.inf, jnp.float32) for _ in range(_U)),
        tuple(jnp.full((_L,), 1.0, jnp.float32) for _ in range(_U)),
        tuple(jnp.zeros((_L,), jnp.int32) for _ in range(_U)),
        tuple(jnp.full((_L,), s * _L, jnp.int32) + lane for s in range(_U)),
        jnp.zeros((_L,), jnp.int32),
    )
    bE, be, bi, _, _ = lax.fori_loop(0, _VOCAB // _XI_CHUNK, chunk_body, init)

    aE, ae, ai = bE[0], be[0], bi[0]
    for s in range(1, _U):
        better = bE[s] * ae < aE * be[s]
        aE = jnp.where(better, bE[s], aE)
        ae = jnp.where(better, be[s], ae)
        ai = jnp.where(better, bi[s], ai)
    sc = aE / ae
    smin = jnp.min(sc)
    return jnp.min(jnp.where(sc == smin, ai, jnp.int32(2**31 - 1)))


def _sc_sample(logits, xi):
    mesh = plsc.VectorSubcoreMesh(core_axis_name="c", subcore_axis_name="s")

    @functools.partial(
        pl.kernel,
        out_type=jax.ShapeDtypeStruct((_BATCH * _L,), jnp.int32),
        mesh=mesh,
        scratch_types=[
            pltpu.VMEM((_VOCAB,), jnp.float32),
            pltpu.VMEM((_HIST,), jnp.float32),
            pltpu.VMEM((_XI_CHUNK,), jnp.float32),
            pltpu.VMEM((_L,), jnp.int32),
        ],
        compiler_params=pltpu.CompilerParams(needs_layout_passes=False),
    )
    def run(logits_hbm, xi_hbm, out_hbm, e_v, hist, xi_buf, tok_v):
        sid = lax.axis_index("s")
        cid = lax.axis_index("c")
        wid = cid * 16 + sid
        r0 = wid * 2

        def zero_body(i, _):
            hist[pl.ds(i * _L, _L)] = jnp.zeros((_L,), jnp.float32)
            return 0

        lax.fori_loop(0, _HIST // _L, zero_body, 0)

        for rr in range(2):
            win = _select_row(r0 + rr, logits_hbm, xi_hbm, e_v, hist, xi_buf)
            tok_v[...] = jnp.full((_L,), win, jnp.int32)
            pltpu.sync_copy(tok_v, out_hbm.at[pl.ds((r0 + rr) * _L, _L)])

    return run(logits.reshape(-1), xi.reshape(-1))


def _tc_finish_body(tok_ref, logits_ref, out_ref):
    col = lax.broadcasted_iota(jnp.int32, (1, _VOCAB), 1)
    blk = pl.program_id(0)
    for j in range(8):
        tok = tok_ref[blk * 8 + j]
        row = logits_ref[pl.ds(j, 1), :]
        out_ref[pl.ds(j, 1), :] = jnp.where(col == tok, row + jnp.float32(50.0), row)


def _tc_finish(logits, tokens):
    return pl.pallas_call(
        _tc_finish_body,
        grid=(_BATCH // 8,),
        in_specs=[
            pl.BlockSpec(memory_space=pltpu.SMEM),
            pl.BlockSpec((8, _VOCAB), lambda i: (i, 0)),
        ],
        out_specs=pl.BlockSpec((8, _VOCAB), lambda i: (i, 0)),
        out_shape=jax.ShapeDtypeStruct((_BATCH, _VOCAB), jnp.float32),
    )(tokens, logits)


def kernel(input_ids, logits, xi):
    del input_ids
    toks = _sc_sample(logits, xi)
    return _tc_finish(logits, toks.reshape(_BATCH, _L)[:, 0])
